# baseline (speedup 1.0000x reference)
import functools

import jax
import jax.numpy as jnp
from jax import lax
from jax.experimental import pallas as pl
from jax.experimental.pallas import tpu as pltpu
from jax.experimental.pallas import tpu_sc as plsc

N, D, E, L = 8192, 1024, 8, 128
NC, NS = 2, 16
NW = NC * NS
CHUNK = N // NW
NVEC = CHUNK // 16
RB = 32
NB_SC = CHUNK // RB
HW = E * 16
TN = 512
NT = N // TN


def _ind_eq(v, e):
    d = v - e
    return jnp.maximum(1 - d * d, 0)


def _ind_ge(v, d):
    return jnp.minimum(jnp.maximum(v - (d - 1), 0), 1)


def _hist_body(idx_hbm, hist_hbm, idx_c, stage, sem):
    wid = lax.axis_index("s") * NC + lax.axis_index("c")
    base = wid * CHUNK
    pltpu.sync_copy(idx_hbm.at[pl.ds(base, CHUNK)], idx_c)
    vecs = [idx_c[pl.ds(t * 16, 16)] for t in range(NVEC)]
    for e in range(E):
        acc = _ind_eq(vecs[0], e)
        for t in range(1, NVEC):
            acc = acc + _ind_eq(vecs[t], e)
        stage[pl.ds(e * 16, 16)] = acc
    pltpu.sync_copy(stage, hist_hbm.at[pl.ds(wid * HW, HW)])


_hist = functools.partial(
    pl.kernel,
    out_type=jax.ShapeDtypeStruct((NW * HW,), jnp.int32),
    mesh=plsc.VectorSubcoreMesh(core_axis_name="c", subcore_axis_name="s"),
    scratch_types=[
        pltpu.VMEM((CHUNK,), jnp.int32),
        pltpu.VMEM((HW,), jnp.int32),
        pltpu.SemaphoreType.DMA,
    ],
)(_hist_body)


def _route_body(x_hbm, idx_hbm, hist_hbm, xs_hbm, pos_hbm, off_hbm,
                idx_c, hist_v, pos_flat, pos_idx, xbuf, off_v, shbuf, sem):
    wid = lax.axis_index("s") * NC + lax.axis_index("c")
    base = wid * CHUNK

    pltpu.sync_copy(idx_hbm.at[pl.ds(base, CHUNK)], idx_c)
    pltpu.sync_copy(hist_hbm, hist_v)

    lane = lax.iota(jnp.int32, 16)

    tot_pl, pri_pl, ch_self = [], [], []
    for e in range(E):
        t_acc = hist_v[pl.ds(e * 16, 16)]
        p_acc = t_acc * 0
        for w in range(1, NW):
            h = hist_v[pl.ds(w * HW + e * 16, 16)]
            t_acc = t_acc + h
            p_acc = p_acc + h * jnp.where(w < wid, 1, 0)
        p_acc = p_acc + hist_v[pl.ds(e * 16, 16)] * jnp.where(wid > 0, 1, 0)
        tot_pl.append(t_acc)
        pri_pl.append(p_acc)
        ch_self.append(idx_c[pl.ds(0, 16)] * 0)

    vecs = [idx_c[pl.ds(t * 16, 16)] for t in range(NVEC)]
    for e in range(E):
        acc = _ind_eq(vecs[0], e)
        for t in range(1, NVEC):
            acc = acc + _ind_eq(vecs[t], e)
        ch_self[e] = acc

    def rotreduce(v):
        for d in (8, 4, 2, 1):
            shbuf[pl.ds(0, 16)] = v
            shbuf[pl.ds(16, 16)] = v
            v = v + shbuf[pl.ds(d, 16)]
        return v

    def exclscan(v):
        inc = v
        for d in (1, 2, 4, 8):
            shbuf[pl.ds(0, 16)] = inc
            shbuf[pl.ds(16, 16)] = inc
            inc = inc + shbuf[pl.ds(16 - d, 16)] * _ind_ge(lane, d)
        return inc - v

    tot_s = [rotreduce(tot_pl[e]) for e in range(E)]
    pri_s = [rotreduce(pri_pl[e]) for e in range(E)]
    ch_x = [exclscan(ch_self[e]) for e in range(E)]

    excl = []
    run = lane * 0
    for e in range(E):
        excl.append(run)
        run = run + tot_s[e]
    bvec = [excl[e] + pri_s[e] + ch_x[e] for e in range(E)]

    @pl.when(wid == 0)
    def _():
        off = run * _ind_eq(lane, E)
        for e in range(E):
            off = off + excl[e] * _ind_eq(lane, e)
        off_v[...] = off
        pltpu.sync_copy(off_v, off_hbm)

    rt = [lane * 0 for _ in range(E)]
    for t in range(NVEC):
        v = vecs[t]
        pos_vec = lane * 0
        for e in range(E):
            mi = _ind_eq(v, e)
            pos_vec = pos_vec + mi * (bvec[e] + rt[e])
            rt[e] = rt[e] + mi
        pos_flat[pl.ds(t * 16, 16)] = pos_vec
        rb, half = t // 2, (t % 2) * 16
        pos_idx[rb, pl.ds(half, 16)] = pos_vec

    pltpu.sync_copy(pos_flat, pos_hbm.at[pl.ds(base, CHUNK)])

    for bb in range(NB_SC):
        pltpu.sync_copy(x_hbm.at[pl.ds(base + bb * RB, RB)], xbuf)
        pltpu.async_copy(xbuf, xs_hbm.at[pos_idx.at[bb]], sem).wait()


_route = functools.partial(
    pl.kernel,
    out_type=[
        jax.ShapeDtypeStruct((N, D), jnp.float32),
        jax.ShapeDtypeStruct((N,), jnp.int32),
        jax.ShapeDtypeStruct((16,), jnp.int32),
    ],
    mesh=plsc.VectorSubcoreMesh(core_axis_name="c", subcore_axis_name="s"),
    scratch_types=[
        pltpu.VMEM((CHUNK,), jnp.int32),
        pltpu.VMEM((NW * HW,), jnp.int32),
        pltpu.VMEM((CHUNK,), jnp.int32),
        pltpu.VMEM((NB_SC, RB), jnp.int32),
        pltpu.VMEM((RB, D), jnp.float32),
        pltpu.VMEM((16,), jnp.int32),
        pltpu.VMEM((32,), jnp.int32),
        pltpu.SemaphoreType.DMA,
    ],
)(_route_body)


def _mm_body(off_ref, x_ref, W_ref, b_ref, Wd_ref, bd_ref, z_ref, y_ref):
    i = pl.program_id(0)
    tlo = i * TN
    x_t = x_ref[...]
    rows = tlo + lax.broadcasted_iota(jnp.int32, (TN, 1), 0)
    z_ref[...] = jnp.zeros((TN, L), jnp.float32)
    for e in range(E):
        lo = off_ref[e]
        hi = off_ref[e + 1] if e < E - 1 else jnp.int32(N)

        @pl.when(jnp.logical_and(hi > tlo, lo < tlo + TN))
        def _():
            ze = jnp.dot(x_t, W_ref[e], preferred_element_type=jnp.float32) + b_ref[e]
            m = jnp.logical_and(rows >= lo, rows < hi)
            z_ref[...] += jnp.where(m, ze, 0.0)
    y_ref[...] = jnp.dot(z_ref[...], Wd_ref[...],
                         preferred_element_type=jnp.float32) + bd_ref[0]


def _grouped_mm(off16, xs, W_enc, b_enc, W_dec, b_dec):
    grid_spec = pltpu.PrefetchScalarGridSpec(
        num_scalar_prefetch=1,
        grid=(NT,),
        in_specs=[
            pl.BlockSpec((TN, D), lambda i, off: (i, 0)),
            pl.BlockSpec((E, D, L), lambda i, off: (0, 0, 0)),
            pl.BlockSpec((E, L), lambda i, off: (0, 0)),
            pl.BlockSpec((L, 1), lambda i, off: (0, 0)),
            pl.BlockSpec((1,), lambda i, off: (0,)),
        ],
        out_specs=[
            pl.BlockSpec((TN, L), lambda i, off: (i, 0)),
            pl.BlockSpec((TN, 1), lambda i, off: (i, 0)),
        ],
    )
    return pl.pallas_call(
        _mm_body,
        grid_spec=grid_spec,
        out_shape=[
            jax.ShapeDtypeStruct((N, L), jnp.float32),
            jax.ShapeDtypeStruct((N, 1), jnp.float32),
        ],
    )(off16, xs, W_enc, b_enc, W_dec, b_dec)


def _combine_body(zs_hbm, ys_hbm, pos_hbm, z_hbm, y_hbm,
                  pos_c, zbuf, ybuf, sem, sem2):
    wid = lax.axis_index("s") * NC + lax.axis_index("c")
    base = wid * CHUNK
    pltpu.sync_copy(pos_hbm.at[pl.ds(base, CHUNK)], pos_c)
    cp = pltpu.async_copy(zs_hbm.at[pos_c], zbuf, sem)
    cpy = pltpu.async_copy(ys_hbm.at[pos_c], ybuf, sem2)
    cp.wait()
    cpy.wait()
    pltpu.sync_copy(zbuf, z_hbm.at[pl.ds(base, CHUNK)])
    pltpu.sync_copy(ybuf, y_hbm.at[pl.ds(base, CHUNK)])


_combine = functools.partial(
    pl.kernel,
    out_type=[
        jax.ShapeDtypeStruct((N, L), jnp.float32),
        jax.ShapeDtypeStruct((N,), jnp.float32),
    ],
    mesh=plsc.VectorSubcoreMesh(core_axis_name="c", subcore_axis_name="s"),
    scratch_types=[
        pltpu.VMEM((CHUNK,), jnp.int32),
        pltpu.VMEM((CHUNK, L), jnp.float32),
        pltpu.VMEM((CHUNK,), jnp.float32),
        pltpu.SemaphoreType.DMA,
        pltpu.SemaphoreType.DMA,
    ],
)(_combine_body)


def kernel(x, individual_idx, W_enc, b_enc, W_dec, b_dec):
    idx = individual_idx.astype(jnp.int32)
    hist = _hist(idx)
    xs, pos, off16 = _route(x, idx, hist)
    zs, ys = _grouped_mm(off16, xs, W_enc, b_enc, W_dec, b_dec)
    z, y = _combine(zs, ys.reshape(N), pos)
    return (y.reshape(N, 1), z)


# speedup vs baseline: 1.0529x; 1.0529x over previous
import functools

import jax
import jax.numpy as jnp
from jax import lax
from jax.experimental import pallas as pl
from jax.experimental.pallas import tpu as pltpu
from jax.experimental.pallas import tpu_sc as plsc

N, D, E, L = 8192, 1024, 8, 128
NC, NS = 2, 16
NW = NC * NS
CHUNK = N // NW
NVEC = CHUNK // 16
RB = 32
NB_SC = CHUNK // RB
HW = E * 16
TN = 512
NT = N // TN


def _ind_eq(v, e):
    d = v - e
    return jnp.maximum(1 - d * d, 0)


def _ind_ge(v, d):
    return jnp.minimum(jnp.maximum(v - (d - 1), 0), 1)


def _hist_body(idx_hbm, hist_hbm, idx_c, stage, sem):
    wid = lax.axis_index("s") * NC + lax.axis_index("c")
    base = wid * CHUNK
    pltpu.sync_copy(idx_hbm.at[pl.ds(base, CHUNK)], idx_c)
    vecs = [idx_c[pl.ds(t * 16, 16)] for t in range(NVEC)]
    for e in range(E):
        acc = _ind_eq(vecs[0], e)
        for t in range(1, NVEC):
            acc = acc + _ind_eq(vecs[t], e)
        stage[pl.ds(e * 16, 16)] = acc
    pltpu.sync_copy(stage, hist_hbm.at[pl.ds(wid * HW, HW)])


_hist = functools.partial(
    pl.kernel,
    out_type=jax.ShapeDtypeStruct((NW * HW,), jnp.int32),
    mesh=plsc.VectorSubcoreMesh(core_axis_name="c", subcore_axis_name="s"),
    scratch_types=[
        pltpu.VMEM((CHUNK,), jnp.int32),
        pltpu.VMEM((HW,), jnp.int32),
        pltpu.SemaphoreType.DMA,
    ],
)(_hist_body)


def _route_body(x_hbm, idx_hbm, hist_hbm, xs_hbm, pos_hbm, off_hbm,
                idx_c, hist_v, pos_flat, pos_idx, xbuf, xbuf2, off_v, shbuf,
                sem, sem_out):
    wid = lax.axis_index("s") * NC + lax.axis_index("c")
    base = wid * CHUNK

    pltpu.sync_copy(idx_hbm.at[pl.ds(base, CHUNK)], idx_c)
    pltpu.sync_copy(hist_hbm, hist_v)

    lane = lax.iota(jnp.int32, 16)

    tot_pl, pri_pl, ch_self = [], [], []
    for e in range(E):
        t_acc = hist_v[pl.ds(e * 16, 16)]
        p_acc = t_acc * 0
        for w in range(1, NW):
            h = hist_v[pl.ds(w * HW + e * 16, 16)]
            t_acc = t_acc + h
            p_acc = p_acc + h * jnp.where(w < wid, 1, 0)
        p_acc = p_acc + hist_v[pl.ds(e * 16, 16)] * jnp.where(wid > 0, 1, 0)
        tot_pl.append(t_acc)
        pri_pl.append(p_acc)
        ch_self.append(idx_c[pl.ds(0, 16)] * 0)

    vecs = [idx_c[pl.ds(t * 16, 16)] for t in range(NVEC)]
    for e in range(E):
        acc = _ind_eq(vecs[0], e)
        for t in range(1, NVEC):
            acc = acc + _ind_eq(vecs[t], e)
        ch_self[e] = acc

    def rotreduce(v):
        for d in (8, 4, 2, 1):
            shbuf[pl.ds(0, 16)] = v
            shbuf[pl.ds(16, 16)] = v
            v = v + shbuf[pl.ds(d, 16)]
        return v

    def exclscan(v):
        inc = v
        for d in (1, 2, 4, 8):
            shbuf[pl.ds(0, 16)] = inc
            shbuf[pl.ds(16, 16)] = inc
            inc = inc + shbuf[pl.ds(16 - d, 16)] * _ind_ge(lane, d)
        return inc - v

    tot_s = [rotreduce(tot_pl[e]) for e in range(E)]
    pri_s = [rotreduce(pri_pl[e]) for e in range(E)]
    ch_x = [exclscan(ch_self[e]) for e in range(E)]

    excl = []
    run = lane * 0
    for e in range(E):
        excl.append(run)
        run = run + tot_s[e]
    bvec = [excl[e] + pri_s[e] + ch_x[e] for e in range(E)]

    @pl.when(wid == 0)
    def _():
        off = run * _ind_eq(lane, E)
        for e in range(E):
            off = off + excl[e] * _ind_eq(lane, e)
        off_v[...] = off
        pltpu.sync_copy(off_v, off_hbm)

    rt = [lane * 0 for _ in range(E)]
    for t in range(NVEC):
        v = vecs[t]
        pos_vec = lane * 0
        for e in range(E):
            mi = _ind_eq(v, e)
            pos_vec = pos_vec + mi * (bvec[e] + rt[e])
            rt[e] = rt[e] + mi
        pos_flat[pl.ds(t * 16, 16)] = pos_vec
        rb, half = t // 2, (t % 2) * 16
        pos_idx[rb, pl.ds(half, 16)] = pos_vec

    pltpu.sync_copy(pos_flat, pos_hbm.at[pl.ds(base, CHUNK)])

    bufs = (xbuf, xbuf2)
    ins = [None] * NB_SC
    outs = [None] * NB_SC
    ins[0] = pltpu.async_copy(x_hbm.at[pl.ds(base, RB)], bufs[0], sem)
    for bb in range(NB_SC):
        if bb >= 1:
            outs[bb - 1].wait()
        if bb + 1 < NB_SC:
            ins[bb + 1] = pltpu.async_copy(
                x_hbm.at[pl.ds(base + (bb + 1) * RB, RB)],
                bufs[(bb + 1) % 2], sem)
        ins[bb].wait()
        outs[bb] = pltpu.async_copy(bufs[bb % 2], xs_hbm.at[pos_idx.at[bb]],
                                    sem_out)
    outs[NB_SC - 1].wait()


_route = functools.partial(
    pl.kernel,
    out_type=[
        jax.ShapeDtypeStruct((N, D), jnp.float32),
        jax.ShapeDtypeStruct((N,), jnp.int32),
        jax.ShapeDtypeStruct((16,), jnp.int32),
    ],
    mesh=plsc.VectorSubcoreMesh(core_axis_name="c", subcore_axis_name="s"),
    scratch_types=[
        pltpu.VMEM((CHUNK,), jnp.int32),
        pltpu.VMEM((NW * HW,), jnp.int32),
        pltpu.VMEM((CHUNK,), jnp.int32),
        pltpu.VMEM((NB_SC, RB), jnp.int32),
        pltpu.VMEM((RB, D), jnp.float32),
        pltpu.VMEM((RB, D), jnp.float32),
        pltpu.VMEM((16,), jnp.int32),
        pltpu.VMEM((32,), jnp.int32),
        pltpu.SemaphoreType.DMA,
        pltpu.SemaphoreType.DMA,
    ],
)(_route_body)


def _mm_body(off_ref, x_ref, W_ref, b_ref, Wd_ref, bd_ref, z_ref, y_ref):
    i = pl.program_id(0)
    tlo = i * TN
    x_t = x_ref[...]
    rows = tlo + lax.broadcasted_iota(jnp.int32, (TN, 1), 0)
    z_ref[...] = jnp.zeros((TN, L), jnp.float32)
    for e in range(E):
        lo = off_ref[e]
        hi = off_ref[e + 1] if e < E - 1 else jnp.int32(N)

        @pl.when(jnp.logical_and(hi > tlo, lo < tlo + TN))
        def _():
            ze = jnp.dot(x_t, W_ref[e], preferred_element_type=jnp.float32) + b_ref[e]
            m = jnp.logical_and(rows >= lo, rows < hi)
            z_ref[...] += jnp.where(m, ze, 0.0)
    y_ref[...] = jnp.dot(z_ref[...], Wd_ref[...],
                         preferred_element_type=jnp.float32) + bd_ref[0]


def _grouped_mm(off16, xs, W_enc, b_enc, W_dec, b_dec):
    grid_spec = pltpu.PrefetchScalarGridSpec(
        num_scalar_prefetch=1,
        grid=(NT,),
        in_specs=[
            pl.BlockSpec((TN, D), lambda i, off: (i, 0)),
            pl.BlockSpec((E, D, L), lambda i, off: (0, 0, 0)),
            pl.BlockSpec((E, L), lambda i, off: (0, 0)),
            pl.BlockSpec((L, 1), lambda i, off: (0, 0)),
            pl.BlockSpec((1,), lambda i, off: (0,)),
        ],
        out_specs=[
            pl.BlockSpec((TN, L), lambda i, off: (i, 0)),
            pl.BlockSpec((TN, 1), lambda i, off: (i, 0)),
        ],
    )
    return pl.pallas_call(
        _mm_body,
        grid_spec=grid_spec,
        out_shape=[
            jax.ShapeDtypeStruct((N, L), jnp.float32),
            jax.ShapeDtypeStruct((N, 1), jnp.float32),
        ],
    )(off16, xs, W_enc, b_enc, W_dec, b_dec)


def _combine_body(zs_hbm, ys_hbm, pos_hbm, z_hbm, y_hbm,
                  pos_c, zbuf, ybuf, sem, sem2):
    wid = lax.axis_index("s") * NC + lax.axis_index("c")
    base = wid * CHUNK
    pltpu.sync_copy(pos_hbm.at[pl.ds(base, CHUNK)], pos_c)
    cp = pltpu.async_copy(zs_hbm.at[pos_c], zbuf, sem)
    cpy = pltpu.async_copy(ys_hbm.at[pos_c], ybuf, sem2)
    cp.wait()
    cpy.wait()
    pltpu.sync_copy(zbuf, z_hbm.at[pl.ds(base, CHUNK)])
    pltpu.sync_copy(ybuf, y_hbm.at[pl.ds(base, CHUNK)])


_combine = functools.partial(
    pl.kernel,
    out_type=[
        jax.ShapeDtypeStruct((N, L), jnp.float32),
        jax.ShapeDtypeStruct((N,), jnp.float32),
    ],
    mesh=plsc.VectorSubcoreMesh(core_axis_name="c", subcore_axis_name="s"),
    scratch_types=[
        pltpu.VMEM((CHUNK,), jnp.int32),
        pltpu.VMEM((CHUNK, L), jnp.float32),
        pltpu.VMEM((CHUNK,), jnp.float32),
        pltpu.SemaphoreType.DMA,
        pltpu.SemaphoreType.DMA,
    ],
)(_combine_body)


def kernel(x, individual_idx, W_enc, b_enc, W_dec, b_dec):
    idx = individual_idx.astype(jnp.int32)
    hist = _hist(idx)
    xs, pos, off16 = _route(x, idx, hist)
    zs, ys = _grouped_mm(off16, xs, W_enc, b_enc, W_dec, b_dec)
    z, y = _combine(zs, ys.reshape(N), pos)
    return (y.reshape(N, 1), z)


# speedup vs baseline: 1.3616x; 1.2932x over previous
import functools

import jax
import jax.numpy as jnp
from jax.experimental import pallas as pl


def _fused_body(idx_ref, x_ref, W_ref, b_ref, Wd_ref, bd_ref, z_ref, y_ref, *, E):
    x_t = x_ref[...]
    ids = idx_ref[...]
    acc = jnp.zeros(z_ref.shape, dtype=jnp.float32)
    for e in range(E):
        ze = jnp.dot(x_t, W_ref[e], preferred_element_type=jnp.float32) + b_ref[e]
        acc = jnp.where(ids == e, ze, acc)
    z_ref[...] = acc
    y_ref[...] = jnp.dot(acc, Wd_ref[...], preferred_element_type=jnp.float32) + bd_ref[0]


def kernel(x, individual_idx, W_enc, b_enc, W_dec, b_dec):
    N, D = x.shape
    E, _, L = W_enc.shape
    TN = 512
    nb = N // TN
    idx2 = individual_idx.astype(jnp.int32).reshape(N, 1)
    xb = x.astype(jnp.bfloat16)
    Wb = W_enc.astype(jnp.bfloat16)

    z, y = pl.pallas_call(
        functools.partial(_fused_body, E=E),
        grid=(nb,),
        in_specs=[
            pl.BlockSpec((TN, 1), lambda i: (i, 0)),
            pl.BlockSpec((TN, D), lambda i: (i, 0)),
            pl.BlockSpec((E, D, L), lambda i: (0, 0, 0)),
            pl.BlockSpec((E, L), lambda i: (0, 0)),
            pl.BlockSpec((L, 1), lambda i: (0, 0)),
            pl.BlockSpec((1,), lambda i: (0,)),
        ],
        out_specs=[
            pl.BlockSpec((TN, L), lambda i: (i, 0)),
            pl.BlockSpec((TN, 1), lambda i: (i, 0)),
        ],
        out_shape=[
            jax.ShapeDtypeStruct((N, L), jnp.float32),
            jax.ShapeDtypeStruct((N, 1), jnp.float32),
        ],
    )(idx2, xb, Wb, b_enc, W_dec, b_dec)
    return (y, z)


# speedup vs baseline: 1.8563x; 1.3633x over previous
import functools

import jax
import jax.numpy as jnp
from jax.experimental import pallas as pl


def _fused_body(idx_ref, x_ref, W_ref, b_ref, Wd_ref, bd_ref, z_ref, y_ref, *, E):
    x_t = x_ref[...].astype(jnp.bfloat16)
    ids = idx_ref[...]
    acc = jnp.zeros(z_ref.shape, dtype=jnp.float32)
    for e in range(E):
        ze = jnp.dot(x_t, W_ref[e].astype(jnp.bfloat16),
                     preferred_element_type=jnp.float32) + b_ref[e]
        acc = jnp.where(ids == e, ze, acc)
    z_ref[...] = acc
    y_ref[...] = jnp.dot(acc, Wd_ref[...], preferred_element_type=jnp.float32) + bd_ref[0]


def kernel(x, individual_idx, W_enc, b_enc, W_dec, b_dec):
    N, D = x.shape
    E, _, L = W_enc.shape
    TN = 512
    nb = N // TN
    idx2 = individual_idx.astype(jnp.int32).reshape(N, 1)

    z, y = pl.pallas_call(
        functools.partial(_fused_body, E=E),
        grid=(nb,),
        in_specs=[
            pl.BlockSpec((TN, 1), lambda i: (i, 0)),
            pl.BlockSpec((TN, D), lambda i: (i, 0)),
            pl.BlockSpec((E, D, L), lambda i: (0, 0, 0)),
            pl.BlockSpec((E, L), lambda i: (0, 0)),
            pl.BlockSpec((L, 1), lambda i: (0, 0)),
            pl.BlockSpec((1,), lambda i: (0,)),
        ],
        out_specs=[
            pl.BlockSpec((TN, L), lambda i: (i, 0)),
            pl.BlockSpec((TN, 1), lambda i: (i, 0)),
        ],
        out_shape=[
            jax.ShapeDtypeStruct((N, L), jnp.float32),
            jax.ShapeDtypeStruct((N, 1), jnp.float32),
        ],
    )(idx2, x, W_enc, b_enc, W_dec, b_dec)
    return (y, z)


# speedup vs baseline: 2.3523x; 1.2672x over previous
import functools

import jax
import jax.numpy as jnp
from jax.experimental import pallas as pl


def _fused_body(idx_ref, x_ref, W_ref, b_ref, Wd_ref, bd_ref, z_ref, y_ref,
                *, E, L):
    x_t = x_ref[...].astype(jnp.bfloat16)
    ids = idx_ref[...]
    big = jnp.dot(x_t, W_ref[...], preferred_element_type=jnp.float32)
    big = big + b_ref[...]
    acc = jnp.zeros(z_ref.shape, dtype=jnp.float32)
    for e in range(E):
        acc = jnp.where(ids == e, big[:, e * L:(e + 1) * L], acc)
    z_ref[...] = acc
    y_ref[...] = jnp.dot(acc, Wd_ref[...], preferred_element_type=jnp.float32) + bd_ref[0]


def kernel(x, individual_idx, W_enc, b_enc, W_dec, b_dec):
    N, D = x.shape
    E, _, L = W_enc.shape
    TN = 512
    nb = N // TN
    idx2 = individual_idx.astype(jnp.int32).reshape(N, 1)
    W_all = W_enc.transpose(1, 0, 2).reshape(D, E * L).astype(jnp.bfloat16)
    b_all = b_enc.reshape(1, E * L)

    z, y = pl.pallas_call(
        functools.partial(_fused_body, E=E, L=L),
        grid=(nb,),
        in_specs=[
            pl.BlockSpec((TN, 1), lambda i: (i, 0)),
            pl.BlockSpec((TN, D), lambda i: (i, 0)),
            pl.BlockSpec((D, E * L), lambda i: (0, 0)),
            pl.BlockSpec((1, E * L), lambda i: (0, 0)),
            pl.BlockSpec((L, 1), lambda i: (0, 0)),
            pl.BlockSpec((1,), lambda i: (0,)),
        ],
        out_specs=[
            pl.BlockSpec((TN, L), lambda i: (i, 0)),
            pl.BlockSpec((TN, 1), lambda i: (i, 0)),
        ],
        out_shape=[
            jax.ShapeDtypeStruct((N, L), jnp.float32),
            jax.ShapeDtypeStruct((N, 1), jnp.float32),
        ],
    )(idx2, x, W_all, b_all, W_dec, b_dec)
    return (y, z)


# speedup vs baseline: 2.6198x; 1.1137x over previous
import functools

import jax
import jax.numpy as jnp
from jax.experimental import pallas as pl


def _fused_body(idx_ref, x_ref, W_ref, b_ref, Wd_ref, bd_ref, z_ref, y_ref,
                *, E, L):
    x_t = x_ref[...].astype(jnp.bfloat16)
    ids = idx_ref[...]
    big = jnp.dot(x_t, W_ref[...], preferred_element_type=jnp.float32)
    big = big + b_ref[...]
    acc = jnp.zeros(z_ref.shape, dtype=jnp.float32)
    for e in range(E):
        acc = jnp.where(ids == e, big[:, e * L:(e + 1) * L], acc)
    z_ref[...] = acc
    y_ref[...] = jnp.dot(acc, Wd_ref[...], preferred_element_type=jnp.float32) + bd_ref[0]


def kernel(x, individual_idx, W_enc, b_enc, W_dec, b_dec):
    N, D = x.shape
    E, _, L = W_enc.shape
    TN = 1024
    nb = N // TN
    idx2 = individual_idx.astype(jnp.int32).reshape(N, 1)
    W_all = W_enc.transpose(1, 0, 2).reshape(D, E * L).astype(jnp.bfloat16)
    b_all = b_enc.reshape(1, E * L)

    z, y = pl.pallas_call(
        functools.partial(_fused_body, E=E, L=L),
        grid=(nb,),
        in_specs=[
            pl.BlockSpec((TN, 1), lambda i: (i, 0)),
            pl.BlockSpec((TN, D), lambda i: (i, 0)),
            pl.BlockSpec((D, E * L), lambda i: (0, 0)),
            pl.BlockSpec((1, E * L), lambda i: (0, 0)),
            pl.BlockSpec((L, 1), lambda i: (0, 0)),
            pl.BlockSpec((1,), lambda i: (0,)),
        ],
        out_specs=[
            pl.BlockSpec((TN, L), lambda i: (i, 0)),
            pl.BlockSpec((TN, 1), lambda i: (i, 0)),
        ],
        out_shape=[
            jax.ShapeDtypeStruct((N, L), jnp.float32),
            jax.ShapeDtypeStruct((N, 1), jnp.float32),
        ],
    )(idx2, x, W_all, b_all, W_dec, b_dec)
    return (y, z)
